# baseline (speedup 1.0000x reference)
import functools

import jax
import jax.numpy as jnp
from jax import lax
from jax.experimental import pallas as pl
from jax.experimental.pallas import tpu as pltpu
from jax.experimental.pallas import tpu_sc as plsc

F32 = jnp.float32
N, C, PP = 3136, 96, 16
NPAD = 3200
GPAD = 3584
TEMP = 0.001
G8 = 8
NROW2 = N * PP


def _unfold_block(x):
    y = x.reshape(C, 4, 56, 4).transpose(2, 1, 3, 0)
    return y.reshape(56, PP, C)


def _pool16(y):
    acc = y[:, 0, :]
    for j in range(1, PP):
        acc = acc + y[:, j, :]
    return acc * (1.0 / PP)


def _layout_body(t_ref, r_ref, a_ref, tf_ref, rf_ref, af_ref, x1_ref, x2_ref):
    yt = _unfold_block(t_ref[...])
    yr = _unfold_block(r_ref[...])
    ya = _unfold_block(a_ref[...])
    tf_ref[...] = yt.reshape(56, PP * C)
    rf_ref[...] = yr.reshape(56, PP * C)
    af_ref[...] = ya.reshape(56, PP * C)
    x1_ref[...] = _pool16(yt)
    x2_ref[...] = _pool16(yr)


def _dist_body(x1_ref, x2_ref, idx_ref):
    x1 = x1_ref[...]
    x2 = x2_ref[...]
    x1n = jnp.sum(x1 * x1, axis=1, keepdims=True)
    x2n = jnp.sum(x2 * x2, axis=1)
    g = lax.dot_general(x1.astype(jnp.bfloat16), x2.astype(jnp.bfloat16),
                        (((1,), (1,)), ((), ())),
                        preferred_element_type=F32)
    d2 = x1n + x2n[None, :] - 2.0 * g
    d = jnp.sqrt(jnp.clip(d2, 1e-30, None))
    rows = lax.broadcasted_iota(jnp.int32, d.shape, 0)
    m = jnp.min(d, axis=0, keepdims=True)
    cand = jnp.where(d <= m, rows, jnp.int32(2**30))
    idx_ref[0, 0, :] = jnp.min(cand, axis=0)


def _combine_body(t_ref, r_ref, a_ref, o_ref):
    T = t_ref[...]
    Rr = r_ref[...]
    A = a_ref[...]
    tn = jnp.sum(T * T, axis=1, keepdims=True)
    rn = jnp.sum(Rr * Rr, axis=1)
    g = lax.dot_general(T, Rr, (((1,), (1,)), ((), ())),
                        preferred_element_type=F32,
                        precision=lax.Precision.HIGHEST)
    e = tn + rn[None, :] - 2.0 * g
    d = jnp.sqrt(jnp.clip(e, 1e-30, None))
    bx = lax.broadcasted_iota(jnp.int32, d.shape, 0) // PP
    by = lax.broadcasted_iota(jnp.int32, d.shape, 1) // PP
    dm = jnp.where(bx == by, d, 1e30)
    z = -dm / TEMP
    mz = jnp.max(z, axis=1, keepdims=True)
    ez = jnp.exp(z - mz)
    s = ez / jnp.sum(ez, axis=1, keepdims=True)
    o_ref[...] = lax.dot_general(s.astype(jnp.bfloat16), A.astype(jnp.bfloat16),
                                 (((1,), (0,)), ((), ())),
                                 preferred_element_type=F32)


def _fold_body(o_ref, img_ref):
    o = o_ref[...].reshape(56, 4, 4, C)
    y = o.transpose(3, 1, 0, 2)
    img_ref[...] = y.reshape(C, 1, 4, 224)


def _make_gather():
    info = plsc.get_sparse_core_info()
    nc = info.num_cores
    bpw = GPAD // (nc * info.num_subcores)
    ch = bpw // 2
    mesh = plsc.VectorSubcoreMesh(core_axis_name="c", subcore_axis_name="s")

    @functools.partial(
        pl.kernel, mesh=mesh,
        out_type=[jax.ShapeDtypeStruct((GPAD, PP * C), F32)] * 2,
        scratch_types=[
            pltpu.VMEM((ch,), jnp.int32),
            pltpu.VMEM((ch,), jnp.int32),
            pltpu.VMEM((ch, PP * C), F32),
            pltpu.SemaphoreType.DMA,
        ],
    )
    def gather_k(rf_hbm, af_hbm, idx_hbm, outr_hbm, outa_hbm,
                 idx_a, idx_b, rows_v, sem):
        wid = lax.axis_index("s") * nc + lax.axis_index("c")
        base = wid * bpw
        pltpu.sync_copy(idx_hbm.at[pl.ds(base, ch)], idx_a)
        pltpu.sync_copy(idx_hbm.at[pl.ds(base + ch, ch)], idx_b)
        for tbl, out in ((rf_hbm, outr_hbm), (af_hbm, outa_hbm)):
            pltpu.async_copy(tbl.at[idx_a], rows_v, sem).wait()
            pltpu.sync_copy(rows_v, out.at[pl.ds(base, ch)])
            pltpu.async_copy(tbl.at[idx_b], rows_v, sem).wait()
            pltpu.sync_copy(rows_v, out.at[pl.ds(base + ch, ch)])

    return gather_k


_gather = _make_gather()


def kernel(target, ref, ref_align):
    tgt4 = target.reshape(C, 56, 4, 224)
    ref4 = ref.reshape(C, 56, 4, 224)
    ra4 = ref_align.reshape(C, 56, 4, 224)

    img_spec = pl.BlockSpec((C, 1, 4, 224), lambda i: (0, i, 0, 0))
    Tf, Rf, Af, x1, x2 = pl.pallas_call(
        _layout_body,
        grid=(56,),
        in_specs=[img_spec, img_spec, img_spec],
        out_specs=[pl.BlockSpec((56, PP * C), lambda i: (i, 0))] * 3 +
                  [pl.BlockSpec((56, C), lambda i: (i, 0))] * 2,
        out_shape=[jax.ShapeDtypeStruct((N, PP * C), F32)] * 3 +
                  [jax.ShapeDtypeStruct((N, C), F32)] * 2,
    )(tgt4, ref4, ra4)

    x2p = jnp.concatenate([x2, jnp.zeros((NPAD - N, C), F32)], axis=0)
    idx3 = pl.pallas_call(
        _dist_body,
        grid=(NPAD // 128,),
        in_specs=[pl.BlockSpec((N, C), lambda i: (0, 0)),
                  pl.BlockSpec((128, C), lambda i: (i, 0))],
        out_specs=pl.BlockSpec((1, 1, 128), lambda i: (i, 0, 0)),
        out_shape=jax.ShapeDtypeStruct((NPAD // 128, 1, 128), jnp.int32),
    )(x1, x2p)
    idxp = jnp.concatenate(
        [idx3.reshape(NPAD), jnp.zeros((GPAD - NPAD,), jnp.int32)], axis=0)

    Rg, Ag = _gather(Rf, Af, idxp)

    O2 = pl.pallas_call(
        _combine_body,
        grid=(N // G8,),
        in_specs=[pl.BlockSpec((G8 * PP, C), lambda i: (i, 0))] * 3,
        out_specs=pl.BlockSpec((G8 * PP, C), lambda i: (i, 0)),
        out_shape=jax.ShapeDtypeStruct((NROW2, C), F32),
    )(Tf.reshape(NROW2, C),
      Rg.reshape(GPAD * PP, C),
      Ag.reshape(GPAD * PP, C))

    out4 = pl.pallas_call(
        _fold_body,
        grid=(56,),
        in_specs=[pl.BlockSpec((56, PP * C), lambda i: (i, 0))],
        out_specs=pl.BlockSpec((C, 1, 4, 224), lambda i: (0, i, 0, 0)),
        out_shape=jax.ShapeDtypeStruct((C, 56, 4, 224), F32),
    )(O2.reshape(N, PP * C))

    return out4.reshape(1, C, 224, 224)


# speedup vs baseline: 2.3775x; 2.3775x over previous
import functools

import jax
import jax.numpy as jnp
from jax import lax
from jax.experimental import pallas as pl
from jax.experimental.pallas import tpu as pltpu
from jax.experimental.pallas import tpu_sc as plsc

F32 = jnp.float32
N, C, PP = 3136, 96, 16
NPAD = 3200
GPAD = 3584
TEMP = 0.001
G8 = 8
NROW2 = N * PP


def _unfold(img):
    return img.reshape(C, 56, 4, 56, 4).transpose(1, 3, 2, 4, 0).reshape(N, PP, C)


def _pool_body(t_ref, r_ref, x1_ref, x2_ref):
    for src, dst in ((t_ref, x1_ref), (r_ref, x2_ref)):
        x = src[...]
        acc = x[:, 0, :]
        for j in range(1, PP):
            acc = acc + x[:, j, :]
        dst[...] = acc * (1.0 / PP)


def _dist_body(x1_ref, x2_ref, idx_ref):
    x1 = x1_ref[...]
    x2 = x2_ref[...]
    x1n = jnp.sum(x1 * x1, axis=1, keepdims=True)
    x2n = jnp.sum(x2 * x2, axis=1)
    g = lax.dot_general(x1.astype(jnp.bfloat16), x2.astype(jnp.bfloat16),
                        (((1,), (1,)), ((), ())),
                        preferred_element_type=F32)
    d2 = x1n + x2n[None, :] - 2.0 * g
    d = jnp.sqrt(jnp.clip(d2, 1e-30, None))
    rows = lax.broadcasted_iota(jnp.int32, d.shape, 0)
    m = jnp.min(d, axis=0, keepdims=True)
    cand = jnp.where(d <= m, rows, jnp.int32(2**30))
    idx_ref[0, 0, :] = jnp.min(cand, axis=0)


def _combine_body(t_ref, r_ref, a_ref, o_ref):
    T = t_ref[...]
    Rr = r_ref[...]
    A = a_ref[...]
    tn = jnp.sum(T * T, axis=1, keepdims=True)
    rn = jnp.sum(Rr * Rr, axis=1)
    g = lax.dot_general(T, Rr, (((1,), (1,)), ((), ())),
                        preferred_element_type=F32,
                        precision=lax.Precision.HIGHEST)
    e = tn + rn[None, :] - 2.0 * g
    d = jnp.sqrt(jnp.clip(e, 1e-30, None))
    bx = lax.broadcasted_iota(jnp.int32, d.shape, 0) // PP
    by = lax.broadcasted_iota(jnp.int32, d.shape, 1) // PP
    dm = jnp.where(bx == by, d, 1e30)
    z = -dm / TEMP
    mz = jnp.max(z, axis=1, keepdims=True)
    ez = jnp.exp(z - mz)
    s = ez / jnp.sum(ez, axis=1, keepdims=True)
    o_ref[...] = lax.dot_general(s.astype(jnp.bfloat16), A.astype(jnp.bfloat16),
                                 (((1,), (0,)), ((), ())),
                                 preferred_element_type=F32)


def _make_gather():
    info = plsc.get_sparse_core_info()
    nc = info.num_cores
    bpw = GPAD // (nc * info.num_subcores)
    ch = bpw // 2
    mesh = plsc.VectorSubcoreMesh(core_axis_name="c", subcore_axis_name="s")

    @functools.partial(
        pl.kernel, mesh=mesh,
        out_type=[jax.ShapeDtypeStruct((GPAD, PP * C), F32)] * 2,
        scratch_types=[
            pltpu.VMEM((ch,), jnp.int32),
            pltpu.VMEM((ch,), jnp.int32),
            pltpu.VMEM((ch, PP * C), F32),
            pltpu.SemaphoreType.DMA,
        ],
    )
    def gather_k(rf_hbm, af_hbm, idx_hbm, outr_hbm, outa_hbm,
                 idx_a, idx_b, rows_v, sem):
        wid = lax.axis_index("s") * nc + lax.axis_index("c")
        base = wid * bpw
        pltpu.sync_copy(idx_hbm.at[pl.ds(base, ch)], idx_a)
        pltpu.sync_copy(idx_hbm.at[pl.ds(base + ch, ch)], idx_b)
        for tbl, out in ((rf_hbm, outr_hbm), (af_hbm, outa_hbm)):
            pltpu.async_copy(tbl.at[idx_a], rows_v, sem).wait()
            pltpu.sync_copy(rows_v, out.at[pl.ds(base, ch)])
            pltpu.async_copy(tbl.at[idx_b], rows_v, sem).wait()
            pltpu.sync_copy(rows_v, out.at[pl.ds(base + ch, ch)])

    return gather_k


_gather = _make_gather()


def kernel(target, ref, ref_align):
    tgt, rf, ra = target[0], ref[0], ref_align[0]
    Tf = _unfold(tgt)
    Rf = _unfold(rf)
    Af = _unfold(ra)

    x1, x2 = pl.pallas_call(
        _pool_body,
        grid=(7,),
        in_specs=[pl.BlockSpec((448, PP, C), lambda i: (i, 0, 0)),
                  pl.BlockSpec((448, PP, C), lambda i: (i, 0, 0))],
        out_specs=[pl.BlockSpec((448, C), lambda i: (i, 0)),
                   pl.BlockSpec((448, C), lambda i: (i, 0))],
        out_shape=[jax.ShapeDtypeStruct((N, C), F32)] * 2,
    )(Tf, Rf)

    x2p = jnp.concatenate([x2, jnp.zeros((NPAD - N, C), F32)], axis=0)
    idx3 = pl.pallas_call(
        _dist_body,
        grid=(NPAD // 128,),
        in_specs=[pl.BlockSpec((N, C), lambda i: (0, 0)),
                  pl.BlockSpec((128, C), lambda i: (i, 0))],
        out_specs=pl.BlockSpec((1, 1, 128), lambda i: (i, 0, 0)),
        out_shape=jax.ShapeDtypeStruct((NPAD // 128, 1, 128), jnp.int32),
    )(x1, x2p)
    idxp = jnp.concatenate(
        [idx3.reshape(NPAD), jnp.zeros((GPAD - NPAD,), jnp.int32)], axis=0)

    Rg, Ag = _gather(Rf.reshape(N, PP * C), Af.reshape(N, PP * C), idxp)

    O2 = pl.pallas_call(
        _combine_body,
        grid=(N // G8,),
        in_specs=[pl.BlockSpec((G8 * PP, C), lambda i: (i, 0))] * 3,
        out_specs=pl.BlockSpec((G8 * PP, C), lambda i: (i, 0)),
        out_shape=jax.ShapeDtypeStruct((NROW2, C), F32),
    )(Tf.reshape(NROW2, C),
      Rg.reshape(GPAD * PP, C),
      Ag.reshape(GPAD * PP, C))

    out = O2.reshape(56, 56, 4, 4, C).transpose(4, 0, 2, 1, 3).reshape(1, C, 224, 224)
    return out
